# baseline (speedup 1.0000x reference)
import jax
import jax.numpy as jnp
from jax.experimental import pallas as pl
from jax.experimental.pallas import tpu as pltpu

_B, _N, _D, _K = 32, 512, 3, 20
_C2, _NCLS = 50, 40
_KPAD = 32


def _manifold_kernel(w1_ref, x_ref, wp_ref, weff_ref, wl3_ref, bl_ref,
                     out_ref, adj_ref, m_ref, a_ref, vm_ref):
    x = x_ref[0]
    colf = jax.lax.broadcasted_iota(
        jnp.int32, (_N, _N), 1).astype(jnp.float32)

    def neg_pairdist(pts):
        inner = jax.lax.dot_general(
            pts, pts, (((1,), (1,)), ((), ())),
            preferred_element_type=jnp.float32)
        sq = jnp.sum(pts * pts, axis=1)
        return (2.0 * inner - sq[:, None]) - sq[None, :]

    def argmax_onehot(a):
        cand = jnp.where(a == vm_ref[...], colf, float(_N))
        cmin = jnp.min(cand, axis=1, keepdims=True)
        return colf == cmin

    def mask_and_rescan(onehot, a):
        an = jnp.where(onehot, -jnp.inf, a)
        adj_ref[...] = an
        vm_ref[...] = jnp.max(an, axis=1, keepdims=True)

    npd = neg_pairdist(x)
    adj_ref[...] = npd
    vm_ref[...] = jnp.max(npd, axis=1, keepdims=True)
    m_ref[...] = jnp.zeros((_N, _N), jnp.float32)

    def body1(k, _):
        a = adj_ref[...]
        onehot = argmax_onehot(a)
        m_ref[...] += onehot.astype(jnp.float32) * w1_ref[k]
        mask_and_rescan(onehot, a)
        return _

    jax.lax.fori_loop(0, _K, body1, None)
    y = jax.lax.dot_general(
        m_ref[...], x, (((1,), (0,)), ((), ())),
        preferred_element_type=jnp.float32)

    npd2 = neg_pairdist(y)
    adj_ref[...] = npd2
    vm_ref[...] = jnp.max(npd2, axis=1, keepdims=True)
    a_ref[...] = jnp.zeros((_KPAD, _N), jnp.float32)
    wp = wp_ref[...]
    krow = jax.lax.broadcasted_iota(jnp.int32, (_KPAD, 1), 0)

    def body2(k, _):
        a = adj_ref[...]
        onehot = argmax_onehot(a)
        arow = jnp.sum(jnp.where(onehot, wp, 0.0), axis=0, keepdims=True)
        a_ref[...] += (krow == k).astype(jnp.float32) * arow
        mask_and_rescan(onehot, a)
        return _

    jax.lax.fori_loop(0, _K, body2, None)

    u = jax.lax.dot_general(
        a_ref[...], y, (((1,), (0,)), ((), ())),
        preferred_element_type=jnp.float32)
    g = jax.lax.dot_general(
        u, weff_ref[...], (((0,), (0,)), ((), ())),
        preferred_element_type=jnp.float32)

    acc = jnp.zeros((1, _NCLS), jnp.float32)
    for d in range(_D):
        acc = acc + jax.lax.dot_general(
            g[d:d + 1, :], wl3_ref[d], (((1,), (0,)), ((), ())),
            preferred_element_type=jnp.float32)
    out_ref[0] = acc + bl_ref[...]


def kernel(inputs, W1_1, W1_2, W2_1, W2_2, Wp, Wl, bl):
    del W1_2
    xp = jnp.pad(inputs, ((0, 0), (0, 0), (0, 8 - _D)))
    w1 = jax.nn.softmax(W1_1[:, 0])
    weff = jax.nn.softmax(W2_1, axis=0) @ jax.nn.softmax(W2_2, axis=0)
    weff_pad = jnp.zeros((_KPAD, _C2), jnp.float32).at[:_K].set(weff)
    wp = jax.nn.softmax(Wp).reshape(_N, 1)
    wl3 = Wl.reshape(_D, _C2, _NCLS)
    bl2 = bl.reshape(1, _NCLS)

    grid_spec = pltpu.PrefetchScalarGridSpec(
        num_scalar_prefetch=0,
        grid=(_B,),
        in_specs=[
            pl.BlockSpec(memory_space=pltpu.SMEM),
            pl.BlockSpec((1, _N, 8), lambda b: (b, 0, 0)),
            pl.BlockSpec((_N, 1), lambda b: (0, 0)),
            pl.BlockSpec((_KPAD, _C2), lambda b: (0, 0)),
            pl.BlockSpec((_D, _C2, _NCLS), lambda b: (0, 0, 0)),
            pl.BlockSpec((1, _NCLS), lambda b: (0, 0)),
        ],
        out_specs=pl.BlockSpec((1, 1, _NCLS), lambda b: (b, 0, 0)),
        scratch_shapes=[
            pltpu.VMEM((_N, _N), jnp.float32),
            pltpu.VMEM((_N, _N), jnp.float32),
            pltpu.VMEM((_KPAD, _N), jnp.float32),
            pltpu.VMEM((_N, 1), jnp.float32),
        ],
    )
    out = pl.pallas_call(
        _manifold_kernel,
        grid_spec=grid_spec,
        out_shape=jax.ShapeDtypeStruct((_B, 1, _NCLS), jnp.float32),
        compiler_params=pltpu.CompilerParams(
            dimension_semantics=("parallel",)),
    )(w1, xp, wp, weff_pad, wl3, bl2)
    return out.reshape(_B, _NCLS)


# speedup vs baseline: 1.7016x; 1.7016x over previous
import jax
import jax.numpy as jnp
from jax.experimental import pallas as pl
from jax.experimental.pallas import tpu as pltpu

_B, _N, _D, _K = 32, 512, 3, 20
_C2, _NCLS = 50, 40
_KPAD = 32


def _manifold_kernel(w1_ref, x_ref, wp_ref, weff_ref, wl3_ref, bl_ref,
                     out_ref, adj_ref, m_ref, a_ref):
    x = x_ref[0]
    colf = jax.lax.broadcasted_iota(
        jnp.int32, (_N, _N), 1).astype(jnp.float32)

    def neg_pairdist(pts):
        inner = jax.lax.dot_general(
            pts, pts, (((1,), (1,)), ((), ())),
            preferred_element_type=jnp.float32)
        sq = jnp.sum(pts * pts, axis=1)
        return (2.0 * inner - sq[:, None]) - sq[None, :]

    def argmax_onehot(a):
        vmax = jnp.max(a, axis=1, keepdims=True)
        cand = jnp.where(a == vmax, colf, float(_N))
        cmin = jnp.min(cand, axis=1, keepdims=True)
        return colf == cmin

    adj_ref[...] = neg_pairdist(x)
    m_ref[...] = jnp.zeros((_N, _N), jnp.float32)

    for k in range(_K):
        a = adj_ref[...]
        onehot = argmax_onehot(a)
        m_ref[...] += onehot.astype(jnp.float32) * w1_ref[k]
        adj_ref[...] = jnp.where(onehot, -jnp.inf, a)

    y = jax.lax.dot_general(
        m_ref[...], x, (((1,), (0,)), ((), ())),
        preferred_element_type=jnp.float32)

    adj_ref[...] = neg_pairdist(y)
    a_ref[...] = jnp.zeros((_KPAD, _N), jnp.float32)
    wp = wp_ref[...]

    for k in range(_K):
        a = adj_ref[...]
        onehot = argmax_onehot(a)
        arow = jnp.sum(jnp.where(onehot, wp, 0.0), axis=0, keepdims=True)
        a_ref[k:k + 1, :] = arow
        adj_ref[...] = jnp.where(onehot, -jnp.inf, a)

    u = jax.lax.dot_general(
        a_ref[...], y, (((1,), (0,)), ((), ())),
        preferred_element_type=jnp.float32)
    g = jax.lax.dot_general(
        u, weff_ref[...], (((0,), (0,)), ((), ())),
        preferred_element_type=jnp.float32)

    acc = jnp.zeros((1, _NCLS), jnp.float32)
    for d in range(_D):
        acc = acc + jax.lax.dot_general(
            g[d:d + 1, :], wl3_ref[d], (((1,), (0,)), ((), ())),
            preferred_element_type=jnp.float32)
    out_ref[0] = acc + bl_ref[...]


def kernel(inputs, W1_1, W1_2, W2_1, W2_2, Wp, Wl, bl):
    del W1_2
    xp = jnp.pad(inputs, ((0, 0), (0, 0), (0, 8 - _D)))
    w1 = jax.nn.softmax(W1_1[:, 0])
    weff = jax.nn.softmax(W2_1, axis=0) @ jax.nn.softmax(W2_2, axis=0)
    weff_pad = jnp.zeros((_KPAD, _C2), jnp.float32).at[:_K].set(weff)
    wp = jax.nn.softmax(Wp).reshape(_N, 1)
    wl3 = Wl.reshape(_D, _C2, _NCLS)
    bl2 = bl.reshape(1, _NCLS)

    grid_spec = pltpu.PrefetchScalarGridSpec(
        num_scalar_prefetch=0,
        grid=(_B,),
        in_specs=[
            pl.BlockSpec(memory_space=pltpu.SMEM),
            pl.BlockSpec((1, _N, 8), lambda b: (b, 0, 0)),
            pl.BlockSpec((_N, 1), lambda b: (0, 0)),
            pl.BlockSpec((_KPAD, _C2), lambda b: (0, 0)),
            pl.BlockSpec((_D, _C2, _NCLS), lambda b: (0, 0, 0)),
            pl.BlockSpec((1, _NCLS), lambda b: (0, 0)),
        ],
        out_specs=pl.BlockSpec((1, 1, _NCLS), lambda b: (b, 0, 0)),
        scratch_shapes=[
            pltpu.VMEM((_N, _N), jnp.float32),
            pltpu.VMEM((_N, _N), jnp.float32),
            pltpu.VMEM((_KPAD, _N), jnp.float32),
        ],
    )
    out = pl.pallas_call(
        _manifold_kernel,
        grid_spec=grid_spec,
        out_shape=jax.ShapeDtypeStruct((_B, 1, _NCLS), jnp.float32),
        compiler_params=pltpu.CompilerParams(
            dimension_semantics=("parallel",)),
    )(w1, xp, wp, weff_pad, wl3, bl2)
    return out.reshape(_B, _NCLS)


# speedup vs baseline: 2.0181x; 1.1860x over previous
import jax
import jax.numpy as jnp
from jax.experimental import pallas as pl
from jax.experimental.pallas import tpu as pltpu

_B, _N, _D, _K = 32, 512, 3, 20
_C2, _NCLS = 50, 40
_KPAD = 32


def _manifold_kernel(w1_ref, x_ref, wp_ref, weff_ref, wl3_ref, bl_ref,
                     colf_ref, out_ref, adj_ref, m_ref, a_ref):
    x = x_ref[0]
    colf = colf_ref[...]

    def neg_pairdist(pts):
        inner = jax.lax.dot_general(
            pts, pts, (((1,), (1,)), ((), ())),
            preferred_element_type=jnp.float32)
        sq = jnp.sum(pts * pts, axis=1)
        return (2.0 * inner - sq[:, None]) - sq[None, :]

    def argmax_onehot(a):
        vmax = jnp.max(a, axis=1, keepdims=True)
        cand = jnp.where(a == vmax, colf, float(_N))
        cmin = jnp.min(cand, axis=1, keepdims=True)
        return colf == cmin

    adj_ref[...] = neg_pairdist(x)

    for k in range(_K):
        a = adj_ref[...]
        onehot = argmax_onehot(a)
        sel = jnp.where(onehot, w1_ref[k], 0.0)
        m_ref[...] = sel if k == 0 else m_ref[...] + sel
        if k < _K - 1:
            adj_ref[...] = jnp.where(onehot, -jnp.inf, a)

    y = jax.lax.dot_general(
        m_ref[...], x, (((1,), (0,)), ((), ())),
        preferred_element_type=jnp.float32)

    adj_ref[...] = neg_pairdist(y)
    a_ref[_K:, :] = jnp.zeros((_KPAD - _K, _N), jnp.float32)
    wp = wp_ref[...]

    for k in range(_K):
        a = adj_ref[...]
        onehot = argmax_onehot(a)
        arow = jnp.sum(jnp.where(onehot, wp, 0.0), axis=0, keepdims=True)
        a_ref[k:k + 1, :] = arow
        if k < _K - 1:
            adj_ref[...] = jnp.where(onehot, -jnp.inf, a)

    u = jax.lax.dot_general(
        a_ref[...], y, (((1,), (0,)), ((), ())),
        preferred_element_type=jnp.float32)
    g = jax.lax.dot_general(
        u, weff_ref[...], (((0,), (0,)), ((), ())),
        preferred_element_type=jnp.float32)

    acc = jnp.zeros((1, _NCLS), jnp.float32)
    for d in range(_D):
        acc = acc + jax.lax.dot_general(
            g[d:d + 1, :], wl3_ref[d], (((1,), (0,)), ((), ())),
            preferred_element_type=jnp.float32)
    out_ref[0] = acc + bl_ref[...]


def kernel(inputs, W1_1, W1_2, W2_1, W2_2, Wp, Wl, bl):
    del W1_2
    xp = jnp.pad(inputs, ((0, 0), (0, 0), (0, 8 - _D)))
    w1 = jax.nn.softmax(W1_1[:, 0])
    weff = jax.nn.softmax(W2_1, axis=0) @ jax.nn.softmax(W2_2, axis=0)
    weff_pad = jnp.zeros((_KPAD, _C2), jnp.float32).at[:_K].set(weff)
    wp = jax.nn.softmax(Wp).reshape(_N, 1)
    wl3 = Wl.reshape(_D, _C2, _NCLS)
    bl2 = bl.reshape(1, _NCLS)
    colf = jnp.broadcast_to(
        jnp.arange(_N, dtype=jnp.float32)[None, :], (_N, _N))

    grid_spec = pltpu.PrefetchScalarGridSpec(
        num_scalar_prefetch=0,
        grid=(_B,),
        in_specs=[
            pl.BlockSpec(memory_space=pltpu.SMEM),
            pl.BlockSpec((1, _N, 8), lambda b: (b, 0, 0)),
            pl.BlockSpec((_N, 1), lambda b: (0, 0)),
            pl.BlockSpec((_KPAD, _C2), lambda b: (0, 0)),
            pl.BlockSpec((_D, _C2, _NCLS), lambda b: (0, 0, 0)),
            pl.BlockSpec((1, _NCLS), lambda b: (0, 0)),
            pl.BlockSpec((_N, _N), lambda b: (0, 0)),
        ],
        out_specs=pl.BlockSpec((1, 1, _NCLS), lambda b: (b, 0, 0)),
        scratch_shapes=[
            pltpu.VMEM((_N, _N), jnp.float32),
            pltpu.VMEM((_N, _N), jnp.float32),
            pltpu.VMEM((_KPAD, _N), jnp.float32),
        ],
    )
    out = pl.pallas_call(
        _manifold_kernel,
        grid_spec=grid_spec,
        out_shape=jax.ShapeDtypeStruct((_B, 1, _NCLS), jnp.float32),
        compiler_params=pltpu.CompilerParams(
            dimension_semantics=("parallel",)),
    )(w1, xp, wp, weff_pad, wl3, bl2, colf)
    return out.reshape(_B, _NCLS)
